# baseline (speedup 1.0000x reference)
import jax
import jax.numpy as jnp
from jax.experimental import pallas as pl
from jax.experimental.pallas import tpu as pltpu

CHUNK = 1024


def _copy_body(table_hbm, out_hbm, buf, in_sems, out_sem):
    batch = out_hbm.shape[0]
    seq_len, d_model = table_hbm.shape
    n = seq_len // CHUNK

    def in_copy(i):
        return pltpu.make_async_copy(
            table_hbm.at[pl.ds(i * CHUNK, CHUNK), :],
            buf.at[pl.ds(i * CHUNK, CHUNK), :], in_sems.at[i])

    def out_copy(i, b):
        return pltpu.make_async_copy(
            buf.at[pl.ds(i * CHUNK, CHUNK), :],
            out_hbm.at[b, pl.ds(i * CHUNK, CHUNK), :], out_sem)

    for i in range(n):
        in_copy(i).start()
    for i in range(n):
        in_copy(i).wait()
        for b in range(batch):
            out_copy(i, b).start()
    for i in range(n):
        for b in range(batch):
            out_copy(i, b).wait()


def kernel(input_ids, pos_table):
    batch, seq_len = input_ids.shape
    d_model = pos_table.shape[1]
    out = pl.pallas_call(
        _copy_body,
        in_specs=[pl.BlockSpec(memory_space=pl.ANY)],
        out_specs=pl.BlockSpec(memory_space=pl.ANY),
        out_shape=jax.ShapeDtypeStruct((batch, seq_len, d_model), pos_table.dtype),
        scratch_shapes=[
            pltpu.VMEM((seq_len, d_model), pos_table.dtype),
            pltpu.SemaphoreType.DMA((seq_len // CHUNK,)),
            pltpu.SemaphoreType.DMA,
        ],
    )(pos_table)
    return out
